# baseline (speedup 1.0000x reference)
import functools

import jax
import jax.numpy as jnp
from jax import lax
from jax.experimental import pallas as pl
from jax.experimental.pallas import tpu as pltpu
from jax.experimental.pallas import tpu_sc as plsc

NC = 2
NS = 16
NW = NC * NS
L = 16
B = 16384
K = 32
BPW = B // NW
SUB = 128
NSUB = BPW // SUB
FPP = 4
NPH = K // FPP

_mesh = plsc.VectorSubcoreMesh(core_axis_name="c", subcore_axis_name="s")


@functools.partial(
    pl.kernel,
    mesh=_mesh,
    compiler_params=pltpu.CompilerParams(use_tc_tiling_on_sc=False),
    out_type=[
        jax.ShapeDtypeStruct((B,), jnp.float32),
        jax.ShapeDtypeStruct((K, B), jnp.float32),
        jax.ShapeDtypeStruct((K, B), jnp.float32),
    ],
    scratch_types=[
        pltpu.VMEM((NSUB, SUB), jnp.int32),
        pltpu.VMEM((NSUB, SUB), jnp.int32),
        pltpu.VMEM((K, BPW), jnp.float32),
        pltpu.VMEM((K, BPW), jnp.float32),
        pltpu.VMEM((BPW,), jnp.float32),
        pltpu.SemaphoreType.DMA,
        pltpu.SemaphoreType.DMA,
        pltpu.SemaphoreType.DMA,
    ],
)
def _mf_sc(uidx_hbm, vidx_hbm, wt_hbm, ht_hbm,
           out_hbm, ue_hbm, ve_hbm,
           uidx_v, vidx_v, u_v, v_v, o_v, sem0, sem1, wsem):
    wid = lax.axis_index("s") * NC + lax.axis_index("c")
    base = wid * BPW

    pltpu.sync_copy(uidx_hbm.at[pl.ds(wid * NSUB, NSUB)], uidx_v)
    pltpu.sync_copy(vidx_hbm.at[pl.ds(wid * NSUB, NSUB)], vidx_v)

    sems = (sem0, sem1)

    def fire_phase(p):
        sem = sems[p % 2]
        copies = []
        for f in range(p * FPP, (p + 1) * FPP):
            for j in range(NSUB):
                copies.append(pltpu.async_copy(
                    wt_hbm.at[f].at[uidx_v.at[j]],
                    u_v.at[f, pl.ds(j * SUB, SUB)], sem))
                copies.append(pltpu.async_copy(
                    ht_hbm.at[f].at[vidx_v.at[j]],
                    v_v.at[f, pl.ds(j * SUB, SUB)], sem))
        return copies

    pending = fire_phase(0)
    for p in range(1, NPH):
        nxt = fire_phase(p)
        for c in pending:
            c.wait()
        pending = nxt
    for c in pending:
        c.wait()

    def group_body(g, carry):
        acc = jnp.zeros((L,), jnp.float32)
        for f in range(K):
            acc = acc + u_v[f, pl.ds(g * L, L)] * v_v[f, pl.ds(g * L, L)]
        o_v[pl.ds(g * L, L)] = 1.0 / (1.0 + jnp.exp(-acc))
        return carry

    lax.fori_loop(0, BPW // L, group_body, 0)

    ue_copy = pltpu.async_copy(u_v, ue_hbm.at[:, pl.ds(base, BPW)], wsem)
    ve_copy = pltpu.async_copy(v_v, ve_hbm.at[:, pl.ds(base, BPW)], wsem)
    pltpu.sync_copy(o_v, out_hbm.at[pl.ds(base, BPW)])
    ue_copy.wait()
    ve_copy.wait()


def kernel(x, W, H):
    uidx = x[:, 0].astype(jnp.int32).reshape(NW * NSUB, SUB)
    vidx = x[:, 1].astype(jnp.int32).reshape(NW * NSUB, SUB)
    out, ue_t, ve_t = _mf_sc(uidx, vidx, W.T, H.T)
    return out, ue_t.T, ve_t.T
